# baseline (speedup 1.0000x reference)
import jax
import jax.numpy as jnp
import numpy as np
from jax.experimental import pallas as pl

_M = 1024
_K = 32768
_R = 16
_TINY = np.float32(np.finfo(np.float32).tiny)


def _rotl(x, r):
    return (x << jnp.uint32(r)) | (x >> jnp.uint32(32 - r))


def _threefry_bits(cnt):
    k0 = jnp.uint32(0)
    k1 = jnp.uint32(42)
    k2 = k0 ^ k1 ^ jnp.uint32(0x1BD11BDA)
    ks = (k0, k1, k2)
    rot = ((13, 15, 26, 6), (17, 29, 16, 24))
    x0 = jnp.zeros_like(cnt) + ks[0]
    x1 = cnt + ks[1]
    for g in range(5):
        for r in rot[g % 2]:
            x0 = x0 + x1
            x1 = _rotl(x1, r)
            x1 = x1 ^ x0
        x0 = x0 + ks[(g + 1) % 3]
        x1 = x1 + ks[(g + 2) % 3] + jnp.uint32(g + 1)
    return x0 ^ x1


def _body(logits_ref, out_ref):
    pid = pl.program_id(0)
    base = (pid * jnp.int32(_R * _K)).astype(jnp.uint32)
    row = jax.lax.broadcasted_iota(jnp.uint32, (_R, _K), 0)
    col_u = jax.lax.broadcasted_iota(jnp.uint32, (_R, _K), 1)
    cnt = base + row * jnp.uint32(_K) + col_u

    bits = _threefry_bits(cnt)
    fb = (bits >> jnp.uint32(9)) | jnp.uint32(0x3F800000)
    f = jax.lax.bitcast_convert_type(fb, jnp.float32) - jnp.float32(1.0)
    u = jnp.maximum(_TINY, f * (jnp.float32(1.0) - _TINY) + _TINY)
    g = -jnp.log(-jnp.log(u))

    p = logits_ref[...] + g
    m = jnp.max(p, axis=1, keepdims=True)
    col = jax.lax.broadcasted_iota(jnp.int32, (_R, _K), 1)
    idx = jnp.min(jnp.where(p == m, col, jnp.int32(_K)), axis=1, keepdims=True)
    out_ref[...] = (col == idx).astype(jnp.float32)


def kernel(logits):
    flat = logits.reshape(_M, _K)
    out = pl.pallas_call(
        _body,
        grid=(_M // _R,),
        in_specs=[pl.BlockSpec((_R, _K), lambda i: (i, 0))],
        out_specs=pl.BlockSpec((_R, _K), lambda i: (i, 0)),
        out_shape=jax.ShapeDtypeStruct((_M, _K), jnp.float32),
    )(flat)
    return out.reshape(logits.shape)
